# baseline (speedup 1.0000x reference)
import functools

import jax
import jax.numpy as jnp
from jax import lax
from jax.experimental import pallas as pl
from jax.experimental.pallas import tpu as pltpu
from jax.experimental.pallas import tpu_sc as plsc

N = 10000
E = 320000
DF = 128
D = 64
G = 64

NW = 32
C = 80
EW = E // NW
K = EW // C
NP = 10112
RT = NP // 16

BN = 2000
GRID = N // BN

_f32 = jnp.float32


@functools.lru_cache(maxsize=None)
def _sc_mesh():
    return plsc.VectorSubcoreMesh(core_axis_name="c", subcore_axis_name="s")


NBUF = 5
SDEPTH = 3


def _edge_body_common(h, ei3, z64, out_p, src_v, dst_v, rowsb, sem_g, sem_s,
                      acc, deg_part):
    cid = lax.axis_index("c")
    sid = lax.axis_index("s")
    wid = cid * 16 + sid
    r0 = sid * RT
    pltpu.sync_copy(z64.at[pl.ds(r0, RT)], acc.at[pl.ds(r0, RT)])
    if deg_part is not None:
        z16, accd, ones_v, out_d, sem_d = deg_part
        pltpu.sync_copy(z16.at[pl.ds(r0, RT)], accd.at[pl.ds(r0, RT)])

        def _fill(i, carry):
            ones_v[i] = jnp.ones((16,), _f32)
            return carry
        lax.fori_loop(0, C, _fill, 0)
    pltpu.sync_copy(ei3.at[0, wid], src_v)
    pltpu.sync_copy(ei3.at[1, wid], dst_v)
    plsc.subcore_barrier()

    for b in range(NBUF - SDEPTH):
        pltpu.async_copy(h.at[src_v.at[b]], rowsb.at[b], sem_g.at[b])

    def _group(t, carry):
        jg = NBUF * t
        for b in range(NBUF):
            j = jg + b
            pltpu.make_async_copy(h.at[src_v.at[0]], rowsb.at[b],
                                  sem_g.at[b]).wait()
            pltpu.async_copy(rowsb.at[b], acc.at[dst_v.at[j]], sem_s.at[b],
                             add=True)
            if deg_part is not None:
                pltpu.async_copy(ones_v, accd.at[dst_v.at[j]], sem_d,
                                 add=True)
            bn = (b + NBUF - SDEPTH) % NBUF
            jn = j + NBUF - SDEPTH

            @pl.when(jnp.logical_and(j >= SDEPTH, jn < K))
            def _():
                pltpu.make_async_copy(rowsb.at[bn], acc.at[dst_v.at[0]],
                                      sem_s.at[bn]).wait()

            @pl.when(jn < K)
            def _():
                pltpu.async_copy(h.at[src_v.at[jn]], rowsb.at[bn],
                                 sem_g.at[bn])
        return carry
    lax.fori_loop(0, K // NBUF, _group, 0)
    for b in range(NBUF):
        pltpu.make_async_copy(rowsb.at[b], acc.at[dst_v.at[0]],
                              sem_s.at[b]).wait()
    if deg_part is not None:
        def _drain(i, carry):
            pltpu.make_async_copy(ones_v, accd.at[dst_v.at[0]], sem_d).wait()
            return carry
        lax.fori_loop(0, K, _drain, 0)
    plsc.subcore_barrier()
    pltpu.sync_copy(acc.at[pl.ds(r0, RT)], out_p.at[cid, pl.ds(r0, RT)])
    if deg_part is not None:
        pltpu.sync_copy(accd.at[pl.ds(r0, RT)],
                        out_d.at[cid, pl.ds(r0, RT)])


@functools.lru_cache(maxsize=None)
def _sc_edge_pass_deg():
    @functools.partial(
        pl.kernel,
        out_type=(jax.ShapeDtypeStruct((2, NP, D), _f32),
                  jax.ShapeDtypeStruct((2, NP, 16), _f32)),
        mesh=_sc_mesh(),
        compiler_params=pltpu.CompilerParams(use_tc_tiling_on_sc=False),
        scratch_types=[
            pltpu.VMEM((K, C), jnp.int32),
            pltpu.VMEM((K, C), jnp.int32),
            pltpu.VMEM((NBUF, C, D), _f32),
            pltpu.VMEM((C, 16), _f32),
            pltpu.VMEM_SHARED((NP, D), _f32),
            pltpu.VMEM_SHARED((NP, 16), _f32),
            pltpu.SemaphoreType.DMA((NBUF,)),
            pltpu.SemaphoreType.DMA((NBUF,)),
            pltpu.SemaphoreType.DMA,
        ],
    )
    def body(h, ei3, z64, z16, out_p, out_d,
             src_v, dst_v, rowsb, ones_v, acc, accd, sem_g, sem_s, sem_d):
        _edge_body_common(h, ei3, z64, out_p, src_v, dst_v, rowsb, sem_g,
                          sem_s, acc, (z16, accd, ones_v, out_d, sem_d))
    return body


@functools.lru_cache(maxsize=None)
def _sc_edge_pass():
    @functools.partial(
        pl.kernel,
        out_type=jax.ShapeDtypeStruct((2, NP, D), _f32),
        mesh=_sc_mesh(),
        compiler_params=pltpu.CompilerParams(use_tc_tiling_on_sc=False),
        scratch_types=[
            pltpu.VMEM((K, C), jnp.int32),
            pltpu.VMEM((K, C), jnp.int32),
            pltpu.VMEM((NBUF, C, D), _f32),
            pltpu.VMEM_SHARED((NP, D), _f32),
            pltpu.SemaphoreType.DMA((NBUF,)),
            pltpu.SemaphoreType.DMA((NBUF,)),
        ],
    )
    def body(h, ei3, z64, out_p, src_v, dst_v, rowsb, acc, sem_g, sem_s):
        _edge_body_common(h, ei3, z64, out_p, src_v, dst_v, rowsb, sem_g,
                          sem_s, acc, None)
    return body


def _norm_rows(v):
    return jnp.maximum(jnp.sqrt(jnp.sum(v * v, axis=1, keepdims=True)), 1e-6)


def _klayer(xb, W):
    nx = jnp.sqrt(jnp.sum(xb * xb, axis=1, keepdims=True))
    xn = xb / jnp.maximum(nx, 1e-6)
    Wn = W / _norm_rows(W)
    lo = lax.dot_general(xn, Wn, (((1,), (1,)), ((), ())),
                         preferred_element_type=_f32)
    return nx * jnp.exp(lo - 1.0)


def _tc_head_body(x_ref, W_ref, o_ref):
    o_ref[...] = _klayer(x_ref[...], W_ref[...])


def _tc_head(x, W_in):
    return pl.pallas_call(
        _tc_head_body,
        grid=(GRID,),
        in_specs=[
            pl.BlockSpec((BN, DF), lambda i: (i, 0)),
            pl.BlockSpec((D, DF), lambda i: (0, 0)),
        ],
        out_specs=pl.BlockSpec((BN, D), lambda i: (i, 0)),
        out_shape=jax.ShapeDtypeStruct((N, D), _f32),
    )(x, W_in)


def _fie_combine(s, deg, h, mu):
    return (s - deg * mu) / jnp.maximum(deg, 1.0) + h


def _tc_mid_body(s_ref, d_ref, h_ref, mu_ref, Wp_ref, o_ref):
    t = _fie_combine(s_ref[...], d_ref[...], h_ref[...], mu_ref[...])
    o_ref[...] = _klayer(t, Wp_ref[...])


def _tc_mid(psum, degsum, h0, mu1, Wp1):
    return pl.pallas_call(
        _tc_mid_body,
        grid=(GRID,),
        in_specs=[
            pl.BlockSpec((BN, D), lambda i: (i, 0)),
            pl.BlockSpec((BN, 1), lambda i: (i, 0)),
            pl.BlockSpec((BN, D), lambda i: (i, 0)),
            pl.BlockSpec((1, D), lambda i: (0, 0)),
            pl.BlockSpec((D, D), lambda i: (0, 0)),
        ],
        out_specs=pl.BlockSpec((BN, D), lambda i: (i, 0)),
        out_shape=jax.ShapeDtypeStruct((N, D), _f32),
    )(psum, degsum, h0, mu1, Wp1)


def _tc_pool_body(s_ref, d_ref, h_ref, mu_ref, b_ref, o_ref, acc_ref):
    i = pl.program_id(0)
    h2 = _fie_combine(s_ref[...], d_ref[...], h_ref[...], mu_ref[...])
    onehot = (b_ref[...] ==
              lax.broadcasted_iota(jnp.int32, (BN, G), 1).astype(_f32)
              ).astype(_f32)
    ext = jnp.concatenate([h2, jnp.ones((BN, D), _f32)], axis=1)
    part = lax.dot_general(onehot, ext, (((0,), (0,)), ((), ())),
                           preferred_element_type=_f32)

    @pl.when(i == 0)
    def _():
        acc_ref[...] = part

    @pl.when(i > 0)
    def _():
        acc_ref[...] += part

    @pl.when(i == pl.num_programs(0) - 1)
    def _():
        a = acc_ref[...]
        o_ref[...] = a[:, :D] / jnp.maximum(a[:, D:D + 1], 1.0)


def _tc_pool(p2sum, degsum, h1, mu2, batchf):
    return pl.pallas_call(
        _tc_pool_body,
        grid=(GRID,),
        in_specs=[
            pl.BlockSpec((BN, D), lambda i: (i, 0)),
            pl.BlockSpec((BN, 1), lambda i: (i, 0)),
            pl.BlockSpec((BN, D), lambda i: (i, 0)),
            pl.BlockSpec((1, D), lambda i: (0, 0)),
            pl.BlockSpec((BN, 1), lambda i: (i, 0)),
        ],
        out_specs=pl.BlockSpec((G, D), lambda i: (0, 0)),
        out_shape=jax.ShapeDtypeStruct((G, D), _f32),
        scratch_shapes=[pltpu.VMEM((G, 2 * D), _f32)],
    )(p2sum, degsum, h1, mu2, batchf)


def kernel(x, edge_index, batch, W_in, mu1, Wp1, mu2):
    ei3 = edge_index.astype(jnp.int32).reshape(2, NW, K, C)
    batchf = batch.astype(_f32).reshape(N, 1)
    z64 = jnp.zeros((NP, D), _f32)
    z16 = jnp.zeros((NP, 16), _f32)

    h0 = _tc_head(x, W_in)
    p, dp = _sc_edge_pass_deg()(h0, ei3, z64, z16)
    psum = p[0, :N] + p[1, :N]
    degsum = dp[0, :N, :1] + dp[1, :N, :1]
    h1 = _tc_mid(psum, degsum, h0, mu1, Wp1)
    p2 = _sc_edge_pass()(h1, ei3, z64)
    p2sum = p2[0, :N] + p2[1, :N]
    pooled = _tc_pool(p2sum, degsum, h1, mu2, batchf)
    weights = jnp.full((G, 1), 1.0 / G, _f32)
    return pooled, weights


# speedup vs baseline: 1.1249x; 1.1249x over previous
import functools

import jax
import jax.numpy as jnp
from jax import lax
from jax.experimental import pallas as pl
from jax.experimental.pallas import tpu as pltpu
from jax.experimental.pallas import tpu_sc as plsc

N = 10000
E = 320000
DF = 128
D = 64
G = 64

NW = 32
C = 80
EW = E // NW
K = EW // C
NP = 10112
RT = NP // 16

BN = 2000
GRID = N // BN

_f32 = jnp.float32


@functools.lru_cache(maxsize=None)
def _sc_mesh():
    return plsc.VectorSubcoreMesh(core_axis_name="c", subcore_axis_name="s")


NBUF = 5
SDEPTH = 2


def _edge_body_common(h, ei3, z64, out_p, src_v, dst_v, rowsb, sem_g, sem_s,
                      acc, deg_part):
    cid = lax.axis_index("c")
    sid = lax.axis_index("s")
    wid = cid * 16 + sid
    r0 = sid * RT
    pltpu.sync_copy(z64.at[pl.ds(r0, RT)], acc.at[pl.ds(r0, RT)])
    if deg_part is not None:
        z16, accd, ones_v, out_d, sem_d = deg_part
        pltpu.sync_copy(z16.at[pl.ds(r0, RT)], accd.at[pl.ds(r0, RT)])

        def _fill(i, carry):
            ones_v[i] = jnp.ones((16,), _f32)
            return carry
        lax.fori_loop(0, C, _fill, 0)
    pltpu.sync_copy(ei3.at[0, wid], src_v)
    pltpu.sync_copy(ei3.at[1, wid], dst_v)
    plsc.subcore_barrier()

    for b in range(NBUF - SDEPTH):
        pltpu.async_copy(h.at[src_v.at[b]], rowsb.at[b], sem_g.at[b])

    def _group(t, carry):
        jg = NBUF * t
        for b in range(NBUF):
            j = jg + b
            pltpu.make_async_copy(h.at[src_v.at[0]], rowsb.at[b],
                                  sem_g.at[b]).wait()
            pltpu.async_copy(rowsb.at[b], acc.at[dst_v.at[j]], sem_s.at[b],
                             add=True)
            if deg_part is not None:
                pltpu.async_copy(ones_v, accd.at[dst_v.at[j]], sem_d,
                                 add=True)
            bn = (b + NBUF - SDEPTH) % NBUF
            jn = j + NBUF - SDEPTH

            @pl.when(jnp.logical_and(j >= SDEPTH, jn < K))
            def _():
                pltpu.make_async_copy(rowsb.at[bn], acc.at[dst_v.at[0]],
                                      sem_s.at[bn]).wait()

            @pl.when(jn < K)
            def _():
                pltpu.async_copy(h.at[src_v.at[jn]], rowsb.at[bn],
                                 sem_g.at[bn])
        return carry
    lax.fori_loop(0, K // NBUF, _group, 0)
    for b in range(NBUF):
        pltpu.make_async_copy(rowsb.at[b], acc.at[dst_v.at[0]],
                              sem_s.at[b]).wait()
    if deg_part is not None:
        def _drain(i, carry):
            pltpu.make_async_copy(ones_v, accd.at[dst_v.at[0]], sem_d).wait()
            return carry
        lax.fori_loop(0, K, _drain, 0)
    plsc.subcore_barrier()
    pltpu.sync_copy(acc.at[pl.ds(r0, RT)], out_p.at[cid, pl.ds(r0, RT)])
    if deg_part is not None:
        pltpu.sync_copy(accd.at[pl.ds(r0, RT)],
                        out_d.at[cid, pl.ds(r0, RT)])


@functools.lru_cache(maxsize=None)
def _sc_edge_pass_deg():
    @functools.partial(
        pl.kernel,
        out_type=(jax.ShapeDtypeStruct((2, NP, D), _f32),
                  jax.ShapeDtypeStruct((2, NP, 16), _f32)),
        mesh=_sc_mesh(),
        compiler_params=pltpu.CompilerParams(use_tc_tiling_on_sc=False),
        scratch_types=[
            pltpu.VMEM((K, C), jnp.int32),
            pltpu.VMEM((K, C), jnp.int32),
            pltpu.VMEM((NBUF, C, D), _f32),
            pltpu.VMEM((C, 16), _f32),
            pltpu.VMEM_SHARED((NP, D), _f32),
            pltpu.VMEM_SHARED((NP, 16), _f32),
            pltpu.SemaphoreType.DMA((NBUF,)),
            pltpu.SemaphoreType.DMA((NBUF,)),
            pltpu.SemaphoreType.DMA,
        ],
    )
    def body(h, ei3, z64, z16, out_p, out_d,
             src_v, dst_v, rowsb, ones_v, acc, accd, sem_g, sem_s, sem_d):
        _edge_body_common(h, ei3, z64, out_p, src_v, dst_v, rowsb, sem_g,
                          sem_s, acc, (z16, accd, ones_v, out_d, sem_d))
    return body


@functools.lru_cache(maxsize=None)
def _sc_edge_pass():
    @functools.partial(
        pl.kernel,
        out_type=jax.ShapeDtypeStruct((2, NP, D), _f32),
        mesh=_sc_mesh(),
        compiler_params=pltpu.CompilerParams(use_tc_tiling_on_sc=False),
        scratch_types=[
            pltpu.VMEM((K, C), jnp.int32),
            pltpu.VMEM((K, C), jnp.int32),
            pltpu.VMEM((NBUF, C, D), _f32),
            pltpu.VMEM_SHARED((NP, D), _f32),
            pltpu.SemaphoreType.DMA((NBUF,)),
            pltpu.SemaphoreType.DMA((NBUF,)),
        ],
    )
    def body(h, ei3, z64, out_p, src_v, dst_v, rowsb, acc, sem_g, sem_s):
        _edge_body_common(h, ei3, z64, out_p, src_v, dst_v, rowsb, sem_g,
                          sem_s, acc, None)
    return body


def _norm_rows(v):
    return jnp.maximum(jnp.sqrt(jnp.sum(v * v, axis=1, keepdims=True)), 1e-6)


def _klayer(xb, W):
    nx = jnp.sqrt(jnp.sum(xb * xb, axis=1, keepdims=True))
    xn = xb / jnp.maximum(nx, 1e-6)
    Wn = W / _norm_rows(W)
    lo = lax.dot_general(xn, Wn, (((1,), (1,)), ((), ())),
                         preferred_element_type=_f32)
    return nx * jnp.exp(lo - 1.0)


def _tc_head_body(x_ref, W_ref, o_ref):
    o_ref[...] = _klayer(x_ref[...], W_ref[...])


def _tc_head(x, W_in):
    return pl.pallas_call(
        _tc_head_body,
        grid=(GRID,),
        in_specs=[
            pl.BlockSpec((BN, DF), lambda i: (i, 0)),
            pl.BlockSpec((D, DF), lambda i: (0, 0)),
        ],
        out_specs=pl.BlockSpec((BN, D), lambda i: (i, 0)),
        out_shape=jax.ShapeDtypeStruct((N, D), _f32),
    )(x, W_in)


def _fie_combine(s, deg, h, mu):
    return (s - deg * mu) / jnp.maximum(deg, 1.0) + h


def _tc_mid_body(s_ref, d_ref, h_ref, mu_ref, Wp_ref, o_ref):
    t = _fie_combine(s_ref[...], d_ref[...], h_ref[...], mu_ref[...])
    o_ref[...] = _klayer(t, Wp_ref[...])


def _tc_mid(psum, degsum, h0, mu1, Wp1):
    return pl.pallas_call(
        _tc_mid_body,
        grid=(GRID,),
        in_specs=[
            pl.BlockSpec((BN, D), lambda i: (i, 0)),
            pl.BlockSpec((BN, 1), lambda i: (i, 0)),
            pl.BlockSpec((BN, D), lambda i: (i, 0)),
            pl.BlockSpec((1, D), lambda i: (0, 0)),
            pl.BlockSpec((D, D), lambda i: (0, 0)),
        ],
        out_specs=pl.BlockSpec((BN, D), lambda i: (i, 0)),
        out_shape=jax.ShapeDtypeStruct((N, D), _f32),
    )(psum, degsum, h0, mu1, Wp1)


def _tc_pool_body(s_ref, d_ref, h_ref, mu_ref, b_ref, o_ref, acc_ref):
    i = pl.program_id(0)
    h2 = _fie_combine(s_ref[...], d_ref[...], h_ref[...], mu_ref[...])
    onehot = (b_ref[...] ==
              lax.broadcasted_iota(jnp.int32, (BN, G), 1).astype(_f32)
              ).astype(_f32)
    ext = jnp.concatenate([h2, jnp.ones((BN, D), _f32)], axis=1)
    part = lax.dot_general(onehot, ext, (((0,), (0,)), ((), ())),
                           preferred_element_type=_f32)

    @pl.when(i == 0)
    def _():
        acc_ref[...] = part

    @pl.when(i > 0)
    def _():
        acc_ref[...] += part

    @pl.when(i == pl.num_programs(0) - 1)
    def _():
        a = acc_ref[...]
        o_ref[...] = a[:, :D] / jnp.maximum(a[:, D:D + 1], 1.0)


def _tc_pool(p2sum, degsum, h1, mu2, batchf):
    return pl.pallas_call(
        _tc_pool_body,
        grid=(GRID,),
        in_specs=[
            pl.BlockSpec((BN, D), lambda i: (i, 0)),
            pl.BlockSpec((BN, 1), lambda i: (i, 0)),
            pl.BlockSpec((BN, D), lambda i: (i, 0)),
            pl.BlockSpec((1, D), lambda i: (0, 0)),
            pl.BlockSpec((BN, 1), lambda i: (i, 0)),
        ],
        out_specs=pl.BlockSpec((G, D), lambda i: (0, 0)),
        out_shape=jax.ShapeDtypeStruct((G, D), _f32),
        scratch_shapes=[pltpu.VMEM((G, 2 * D), _f32)],
    )(p2sum, degsum, h1, mu2, batchf)


def kernel(x, edge_index, batch, W_in, mu1, Wp1, mu2):
    ei3 = edge_index.astype(jnp.int32).reshape(2, NW, K, C)
    batchf = batch.astype(_f32).reshape(N, 1)
    z64 = jnp.zeros((NP, D), _f32)
    z16 = jnp.zeros((NP, 16), _f32)

    h0 = _tc_head(x, W_in)
    p, dp = _sc_edge_pass_deg()(h0, ei3, z64, z16)
    psum = p[0, :N] + p[1, :N]
    degsum = dp[0, :N, :1] + dp[1, :N, :1]
    h1 = _tc_mid(psum, degsum, h0, mu1, Wp1)
    p2 = _sc_edge_pass()(h1, ei3, z64)
    p2sum = p2[0, :N] + p2[1, :N]
    pooled = _tc_pool(p2sum, degsum, h1, mu2, batchf)
    weights = jnp.full((G, 1), 1.0 / G, _f32)
    return pooled, weights


# speedup vs baseline: 1.2047x; 1.0710x over previous
import functools

import jax
import jax.numpy as jnp
from jax import lax
from jax.experimental import pallas as pl
from jax.experimental.pallas import tpu as pltpu
from jax.experimental.pallas import tpu_sc as plsc

N = 10000
E = 320000
DF = 128
D = 64
G = 64

NW = 32
C = 80
EW = E // NW
K = EW // C
NP = 10112
RT = NP // 16

BN = 2000
GRID = N // BN

_f32 = jnp.float32


@functools.lru_cache(maxsize=None)
def _sc_mesh():
    return plsc.VectorSubcoreMesh(core_axis_name="c", subcore_axis_name="s")


NBUF = 5
SDEPTH = 2


def _edge_body_common(h, ei3, z64, out_p, src_v, dst_v, rowsb, sem_g, sem_s,
                      acc, deg_part):
    cid = lax.axis_index("c")
    sid = lax.axis_index("s")
    wid = cid * 16 + sid
    r0 = sid * RT
    pltpu.sync_copy(z64.at[pl.ds(r0, RT)], acc.at[pl.ds(r0, RT)])
    if deg_part is not None:
        z16, accd, ones_v, out_d, sem_d = deg_part
        pltpu.sync_copy(z16.at[pl.ds(r0, RT)], accd.at[pl.ds(r0, RT)])

        def _fill(i, carry):
            ones_v[i] = jnp.ones((16,), _f32)
            return carry
        lax.fori_loop(0, C, _fill, 0)
    pltpu.sync_copy(ei3.at[0, wid], src_v)
    pltpu.sync_copy(ei3.at[1, wid], dst_v)
    plsc.subcore_barrier()

    for b in range(NBUF - SDEPTH):
        pltpu.async_copy(h.at[src_v.at[b]], rowsb.at[b], sem_g.at[b])

    def _group(t, carry):
        jg = NBUF * t
        for b in range(NBUF):
            j = jg + b
            pltpu.make_async_copy(h.at[src_v.at[0]], rowsb.at[b],
                                  sem_g.at[b]).wait()
            pltpu.async_copy(rowsb.at[b], acc.at[dst_v.at[j]], sem_s.at[b],
                             add=True)
            if deg_part is not None:
                pltpu.async_copy(ones_v, accd.at[dst_v.at[j]], sem_d,
                                 add=True)
            bn = (b + NBUF - SDEPTH) % NBUF
            jn = j + NBUF - SDEPTH

            @pl.when(jnp.logical_and(j >= SDEPTH, jn < K))
            def _():
                pltpu.make_async_copy(rowsb.at[bn], acc.at[dst_v.at[0]],
                                      sem_s.at[bn]).wait()

            @pl.when(jn < K)
            def _():
                pltpu.async_copy(h.at[src_v.at[jn]], rowsb.at[bn],
                                 sem_g.at[bn])
        return carry
    lax.fori_loop(0, K // NBUF, _group, 0)
    for b in range(NBUF):
        pltpu.make_async_copy(rowsb.at[b], acc.at[dst_v.at[0]],
                              sem_s.at[b]).wait()
    if deg_part is not None:
        def _drain(i, carry):
            pltpu.make_async_copy(ones_v, accd.at[dst_v.at[0]], sem_d).wait()
            return carry
        lax.fori_loop(0, K, _drain, 0)
    plsc.subcore_barrier()
    pltpu.sync_copy(acc.at[pl.ds(r0, RT)], out_p.at[cid, pl.ds(r0, RT)])
    if deg_part is not None:
        pltpu.sync_copy(accd.at[pl.ds(r0, RT)],
                        out_d.at[cid, pl.ds(r0, RT)])


@functools.lru_cache(maxsize=None)
def _sc_edge_pass_deg():
    @functools.partial(
        pl.kernel,
        out_type=(jax.ShapeDtypeStruct((2, NP, D), _f32),
                  jax.ShapeDtypeStruct((2, NP, 16), _f32)),
        mesh=_sc_mesh(),
        compiler_params=pltpu.CompilerParams(use_tc_tiling_on_sc=False),
        scratch_types=[
            pltpu.VMEM((K, C), jnp.int32),
            pltpu.VMEM((K, C), jnp.int32),
            pltpu.VMEM((NBUF, C, D), _f32),
            pltpu.VMEM((C, 16), _f32),
            pltpu.VMEM_SHARED((NP, D), _f32),
            pltpu.VMEM_SHARED((NP, 16), _f32),
            pltpu.SemaphoreType.DMA((NBUF,)),
            pltpu.SemaphoreType.DMA((NBUF,)),
            pltpu.SemaphoreType.DMA,
        ],
    )
    def body(h, ei3, z64, z16, out_p, out_d,
             src_v, dst_v, rowsb, ones_v, acc, accd, sem_g, sem_s, sem_d):
        _edge_body_common(h, ei3, z64, out_p, src_v, dst_v, rowsb, sem_g,
                          sem_s, acc, (z16, accd, ones_v, out_d, sem_d))
    return body


@functools.lru_cache(maxsize=None)
def _sc_edge_pass():
    @functools.partial(
        pl.kernel,
        out_type=jax.ShapeDtypeStruct((2, NP, D), _f32),
        mesh=_sc_mesh(),
        compiler_params=pltpu.CompilerParams(use_tc_tiling_on_sc=False),
        scratch_types=[
            pltpu.VMEM((K, C), jnp.int32),
            pltpu.VMEM((K, C), jnp.int32),
            pltpu.VMEM((NBUF, C, D), _f32),
            pltpu.VMEM_SHARED((NP, D), _f32),
            pltpu.SemaphoreType.DMA((NBUF,)),
            pltpu.SemaphoreType.DMA((NBUF,)),
        ],
    )
    def body(h, ei3, z64, out_p, src_v, dst_v, rowsb, acc, sem_g, sem_s):
        _edge_body_common(h, ei3, z64, out_p, src_v, dst_v, rowsb, sem_g,
                          sem_s, acc, None)
    return body


def _norm_rows(v):
    return jnp.maximum(jnp.sqrt(jnp.sum(v * v, axis=1, keepdims=True)), 1e-6)


def _klayer(xb, W):
    nx = jnp.sqrt(jnp.sum(xb * xb, axis=1, keepdims=True))
    xn = xb / jnp.maximum(nx, 1e-6)
    Wn = W / _norm_rows(W)
    lo = lax.dot_general(xn, Wn, (((1,), (1,)), ((), ())),
                         preferred_element_type=_f32)
    return nx * jnp.exp(lo - 1.0)


def _tc_head_body(x_ref, W_ref, o_ref):
    o_ref[...] = _klayer(x_ref[...], W_ref[...])


def _tc_head(x, W_in):
    return pl.pallas_call(
        _tc_head_body,
        grid=(GRID,),
        in_specs=[
            pl.BlockSpec((BN, DF), lambda i: (i, 0)),
            pl.BlockSpec((D, DF), lambda i: (0, 0)),
        ],
        out_specs=pl.BlockSpec((BN, D), lambda i: (i, 0)),
        out_shape=jax.ShapeDtypeStruct((N, D), _f32),
    )(x, W_in)


def _fie_combine(p0, p1, d0, d1, h, mu):
    deg = (d0 + d1)[:, :1]
    s = p0 + p1
    return (s - deg * mu) / jnp.maximum(deg, 1.0) + h


_P_SPECS = [
    pl.BlockSpec((1, BN, D), lambda i: (0, i, 0)),
    pl.BlockSpec((1, BN, D), lambda i: (1, i, 0)),
    pl.BlockSpec((1, BN, 16), lambda i: (0, i, 0)),
    pl.BlockSpec((1, BN, 16), lambda i: (1, i, 0)),
]


def _tc_mid_body(p0_ref, p1_ref, d0_ref, d1_ref, h_ref, mu_ref, Wp_ref,
                 o_ref):
    t = _fie_combine(p0_ref[0], p1_ref[0], d0_ref[0], d1_ref[0],
                     h_ref[...], mu_ref[...])
    o_ref[...] = _klayer(t, Wp_ref[...])


def _tc_mid(p, dp, h0, mu1, Wp1):
    return pl.pallas_call(
        _tc_mid_body,
        grid=(GRID,),
        in_specs=_P_SPECS + [
            pl.BlockSpec((BN, D), lambda i: (i, 0)),
            pl.BlockSpec((1, D), lambda i: (0, 0)),
            pl.BlockSpec((D, D), lambda i: (0, 0)),
        ],
        out_specs=pl.BlockSpec((BN, D), lambda i: (i, 0)),
        out_shape=jax.ShapeDtypeStruct((N, D), _f32),
    )(p, p, dp, dp, h0, mu1, Wp1)


def _tc_pool_body(p0_ref, p1_ref, d0_ref, d1_ref, h_ref, mu_ref, b_ref,
                  o_ref, acc_ref):
    i = pl.program_id(0)
    h2 = _fie_combine(p0_ref[0], p1_ref[0], d0_ref[0], d1_ref[0],
                      h_ref[...], mu_ref[...])
    onehot = (b_ref[...] ==
              lax.broadcasted_iota(jnp.int32, (BN, G), 1).astype(_f32)
              ).astype(_f32)
    ext = jnp.concatenate([h2, jnp.ones((BN, D), _f32)], axis=1)
    part = lax.dot_general(onehot, ext, (((0,), (0,)), ((), ())),
                           preferred_element_type=_f32)

    @pl.when(i == 0)
    def _():
        acc_ref[...] = part

    @pl.when(i > 0)
    def _():
        acc_ref[...] += part

    @pl.when(i == pl.num_programs(0) - 1)
    def _():
        a = acc_ref[...]
        o_ref[...] = a[:, :D] / jnp.maximum(a[:, D:D + 1], 1.0)


def _tc_pool(p, dp, h1, mu2, batchf):
    return pl.pallas_call(
        _tc_pool_body,
        grid=(GRID,),
        in_specs=_P_SPECS + [
            pl.BlockSpec((BN, D), lambda i: (i, 0)),
            pl.BlockSpec((1, D), lambda i: (0, 0)),
            pl.BlockSpec((BN, 1), lambda i: (i, 0)),
        ],
        out_specs=pl.BlockSpec((G, D), lambda i: (0, 0)),
        out_shape=jax.ShapeDtypeStruct((G, D), _f32),
        scratch_shapes=[pltpu.VMEM((G, 2 * D), _f32)],
    )(p, p, dp, dp, h1, mu2, batchf)


def kernel(x, edge_index, batch, W_in, mu1, Wp1, mu2):
    ei3 = edge_index.astype(jnp.int32).reshape(2, NW, K, C)
    batchf = batch.astype(_f32).reshape(N, 1)
    z64 = jnp.zeros((NP, D), _f32)
    z16 = jnp.zeros((NP, 16), _f32)

    h0 = _tc_head(x, W_in)
    p, dp = _sc_edge_pass_deg()(h0, ei3, z64, z16)
    h1 = _tc_mid(p, dp, h0, mu1, Wp1)
    p2 = _sc_edge_pass()(h1, ei3, z64)
    pooled = _tc_pool(p2, dp, h1, mu2, batchf)
    weights = jnp.full((G, 1), 1.0 / G, _f32)
    return pooled, weights
